# Initial kernel scaffold; baseline (speedup 1.0000x reference)
import jax
import jax.numpy as jnp
from jax.experimental import pallas as pl


def kernel(x, edge_index, W_msg, W_self):
    raise NotImplementedError("write your pallas kernel here")


# speedup vs baseline: 4.6879x; 4.6879x over previous
import jax
import jax.numpy as jnp
from jax import lax
from jax.experimental import pallas as pl
from jax.experimental.pallas import tpu as pltpu
from jax.experimental.pallas import tpu_sc as plsc

N = 10000
D = 128
E = 320000

NC = 2
NS = 16
NW = NC * NS
E_PER_W = E // NW
BLK = 80
NBLK = E_PER_W // BLK
ROWS_PER_SUB = 640
N_PAD = NS * ROWS_PER_SUB
INIT_CHUNKS = ROWS_PER_SUB // BLK


def _sc_body(x_hbm, src_hbm, dst_hbm, zrows_hbm, ones_hbm,
             pagg_hbm, pdeg_hbm,
             src_v, dst_v, rows_v, ones_v, agg_sh, sem):
    c = lax.axis_index("c")
    s = lax.axis_index("s")
    wid = s * NC + c
    r0 = s * ROWS_PER_SUB
    base = wid * E_PER_W

    pltpu.sync_copy(ones_hbm, ones_v)

    def zero_acc():
        pltpu.sync_copy(zrows_hbm, rows_v)
        for k in range(INIT_CHUNKS):
            pltpu.sync_copy(rows_v, agg_sh.at[pl.ds(r0 + k * BLK, BLK)])

    def write_out(out_hbm):
        for k in range(INIT_CHUNKS):
            row = r0 + k * BLK
            pltpu.sync_copy(agg_sh.at[pl.ds(row, BLK)], rows_v)
            pltpu.sync_copy(rows_v, out_hbm.at[c, pl.ds(row, BLK)])

    zero_acc()
    plsc.subcore_barrier()

    def deg_body(i, carry):
        pltpu.sync_copy(dst_hbm.at[pl.ds(base + i * BLK, BLK)], dst_v)
        pltpu.sync_copy(ones_v, agg_sh.at[dst_v], add=True)
        return carry

    lax.fori_loop(0, NBLK, deg_body, 0)
    plsc.subcore_barrier()
    write_out(pdeg_hbm)
    zero_acc()
    plsc.subcore_barrier()

    def agg_body(i, carry):
        off = base + i * BLK
        pltpu.sync_copy(src_hbm.at[pl.ds(off, BLK)], src_v)
        pltpu.sync_copy(dst_hbm.at[pl.ds(off, BLK)], dst_v)
        pltpu.async_copy(x_hbm.at[src_v], rows_v, sem).wait()
        pltpu.sync_copy(rows_v, agg_sh.at[dst_v], add=True)
        return carry

    lax.fori_loop(0, NBLK, agg_body, 0)
    plsc.subcore_barrier()
    write_out(pagg_hbm)


_sc_scatter = pl.kernel(
    _sc_body,
    out_type=[
        jax.ShapeDtypeStruct((NC, N_PAD, D), jnp.float32),
        jax.ShapeDtypeStruct((NC, N_PAD, D), jnp.float32),
    ],
    mesh=plsc.VectorSubcoreMesh(core_axis_name="c", subcore_axis_name="s"),
    scratch_types=[
        pltpu.VMEM((BLK,), jnp.int32),
        pltpu.VMEM((BLK,), jnp.int32),
        pltpu.VMEM((BLK, D), jnp.float32),
        pltpu.VMEM((BLK, D), jnp.float32),
        pltpu.VMEM_SHARED((N_PAD, D), jnp.float32),
        pltpu.SemaphoreType.DMA,
    ],
)


ROWS_PER_STEP = 1000
GRID = N // ROWS_PER_STEP


def _tc_body(pagg0, pagg1, pdeg0, pdeg1, x, wm, ws, out):
    i = pl.program_id(0)
    agg = pagg0[...] + pagg1[...]
    deg = pdeg0[...] + pdeg1[...]
    aggn = agg / jnp.maximum(deg, 1.0)
    h = jnp.dot(aggn, wm[...], preferred_element_type=jnp.float32)
    h = h + jnp.dot(x[...], ws[...], preferred_element_type=jnp.float32)
    h = jnp.maximum(h, 0.0)
    part = jnp.sum(h, axis=0, keepdims=True) * jnp.float32(1.0 / N)

    @pl.when(i == 0)
    def _():
        out[...] = part

    @pl.when(i > 0)
    def _():
        out[...] += part


_tc_combine = pl.pallas_call(
    _tc_body,
    grid=(GRID,),
    in_specs=[
        pl.BlockSpec((ROWS_PER_STEP, D), lambda i: (i, 0)),
        pl.BlockSpec((ROWS_PER_STEP, D), lambda i: (i, 0)),
        pl.BlockSpec((ROWS_PER_STEP, D), lambda i: (i, 0)),
        pl.BlockSpec((ROWS_PER_STEP, D), lambda i: (i, 0)),
        pl.BlockSpec((ROWS_PER_STEP, D), lambda i: (i, 0)),
        pl.BlockSpec((D, D), lambda i: (0, 0)),
        pl.BlockSpec((D, D), lambda i: (0, 0)),
    ],
    out_specs=pl.BlockSpec((1, D), lambda i: (0, 0)),
    out_shape=jax.ShapeDtypeStruct((1, D), jnp.float32),
)


@jax.jit
def kernel(x, edge_index, W_msg, W_self):
    src = edge_index[0]
    dst = edge_index[1]
    zrows = jnp.zeros((BLK, D), jnp.float32)
    ones = jnp.ones((BLK, D), jnp.float32)
    pagg, pdeg = _sc_scatter(x, src, dst, zrows, ones)
    pooled = _tc_combine(pagg[0, :N], pagg[1, :N], pdeg[0, :N], pdeg[1, :N],
                         x, W_msg, W_self)
    return pooled.reshape(D)


# speedup vs baseline: 8.4303x; 1.7983x over previous
import jax
import jax.numpy as jnp
from jax import lax
from jax.experimental import pallas as pl
from jax.experimental.pallas import tpu as pltpu
from jax.experimental.pallas import tpu_sc as plsc

N = 10000
D = 128
E = 320000

NC = 2
NS = 16
NW = NC * NS
E_PER_W = E // NW
BLK = 80
NBLK = E_PER_W // BLK
ROWS_PER_SUB = 640
N_PAD = NS * ROWS_PER_SUB
INIT_CHUNKS = ROWS_PER_SUB // BLK


NBLK_PAD = 128


def _sc_body(x_hbm, e4_hbm, zrows_hbm, ones_hbm,
             pagg_hbm, pdeg_hbm,
             ib0, ib1, ib2, ib3, rows0, rows1, agg_sh,
             sem0, sem1, sem2, sem3, sem4, sem5):
    c = lax.axis_index("c")
    s = lax.axis_index("s")
    wid = s * NC + c
    r0 = s * ROWS_PER_SUB

    def ib_wait(ib, sem):
        pltpu.make_async_copy(e4_hbm.at[wid, 0], ib, sem).wait()

    def row_wait(rows, sem):
        pltpu.make_async_copy(zrows_hbm, rows, sem).wait()

    def zero_acc():
        pltpu.sync_copy(zrows_hbm, rows0)
        for k in range(INIT_CHUNKS):
            pltpu.sync_copy(rows0, agg_sh.at[pl.ds(r0 + k * BLK, BLK)])

    def write_out(out_hbm):
        for k in range(INIT_CHUNKS):
            row = r0 + k * BLK
            pltpu.sync_copy(agg_sh.at[pl.ds(row, BLK)], rows0)
            pltpu.sync_copy(rows0, out_hbm.at[c, pl.ds(row, BLK)])

    pltpu.sync_copy(ones_hbm, rows1)
    zero_acc()
    plsc.subcore_barrier()
    pltpu.async_copy(e4_hbm.at[wid, 0], ib0, sem2)
    pltpu.async_copy(e4_hbm.at[wid, 1], ib1, sem3)

    def deg_pair(p, carry):
        a = 2 * p
        ib_wait(ib0, sem2)
        pltpu.sync_copy(rows1, agg_sh.at[ib0.at[1]], add=True)
        pltpu.async_copy(e4_hbm.at[wid, a + 2], ib0, sem2)
        ib_wait(ib1, sem3)
        pltpu.sync_copy(rows1, agg_sh.at[ib1.at[1]], add=True)
        pltpu.async_copy(e4_hbm.at[wid, a + 3], ib1, sem3)
        return carry

    lax.fori_loop(0, (NBLK - 1) // 2, deg_pair, 0)
    ib_wait(ib0, sem2)
    pltpu.sync_copy(rows1, agg_sh.at[ib0.at[1]], add=True)
    ib_wait(ib1, sem3)
    plsc.subcore_barrier()
    write_out(pdeg_hbm)
    zero_acc()
    plsc.subcore_barrier()

    pltpu.async_copy(e4_hbm.at[wid, 0], ib0, sem2)
    pltpu.async_copy(e4_hbm.at[wid, 1], ib1, sem3)
    pltpu.async_copy(e4_hbm.at[wid, 2], ib2, sem4)
    pltpu.async_copy(e4_hbm.at[wid, 3], ib3, sem5)
    ib_wait(ib0, sem2)
    pltpu.async_copy(x_hbm.at[ib0.at[0]], rows0, sem0)

    def quad_body(q, carry):
        m = 4 * q
        ib_wait(ib1, sem3)
        pltpu.async_copy(x_hbm.at[ib1.at[0]], rows1, sem1)
        row_wait(rows0, sem0)
        pltpu.sync_copy(rows0, agg_sh.at[ib0.at[1]], add=True)
        pltpu.async_copy(e4_hbm.at[wid, m + 4], ib0, sem2)
        ib_wait(ib2, sem4)
        pltpu.async_copy(x_hbm.at[ib2.at[0]], rows0, sem0)
        row_wait(rows1, sem1)
        pltpu.sync_copy(rows1, agg_sh.at[ib1.at[1]], add=True)
        pltpu.async_copy(e4_hbm.at[wid, m + 5], ib1, sem3)
        ib_wait(ib3, sem5)
        pltpu.async_copy(x_hbm.at[ib3.at[0]], rows1, sem1)
        row_wait(rows0, sem0)
        pltpu.sync_copy(rows0, agg_sh.at[ib2.at[1]], add=True)
        pltpu.async_copy(e4_hbm.at[wid, m + 6], ib2, sem4)
        row_wait(rows1, sem1)
        pltpu.sync_copy(rows1, agg_sh.at[ib3.at[1]], add=True)
        pltpu.async_copy(e4_hbm.at[wid, m + 7], ib3, sem5)
        ib_wait(ib0, sem2)
        pltpu.async_copy(x_hbm.at[ib0.at[0]], rows0, sem0)
        return carry

    lax.fori_loop(0, NBLK // 4, quad_body, 0)
    row_wait(rows0, sem0)
    pltpu.sync_copy(rows0, agg_sh.at[ib0.at[1]], add=True)
    ib_wait(ib1, sem3)
    ib_wait(ib2, sem4)
    ib_wait(ib3, sem5)
    plsc.subcore_barrier()
    write_out(pagg_hbm)


_sc_scatter = pl.kernel(
    _sc_body,
    out_type=[
        jax.ShapeDtypeStruct((NC, N_PAD, D), jnp.float32),
        jax.ShapeDtypeStruct((NC, N_PAD, D), jnp.float32),
    ],
    mesh=plsc.VectorSubcoreMesh(core_axis_name="c", subcore_axis_name="s"),
    scratch_types=[
        pltpu.VMEM((2, BLK), jnp.int32),
        pltpu.VMEM((2, BLK), jnp.int32),
        pltpu.VMEM((2, BLK), jnp.int32),
        pltpu.VMEM((2, BLK), jnp.int32),
        pltpu.VMEM((BLK, D), jnp.float32),
        pltpu.VMEM((BLK, D), jnp.float32),
        pltpu.VMEM_SHARED((N_PAD, D), jnp.float32),
        pltpu.SemaphoreType.DMA,
        pltpu.SemaphoreType.DMA,
        pltpu.SemaphoreType.DMA,
        pltpu.SemaphoreType.DMA,
        pltpu.SemaphoreType.DMA,
        pltpu.SemaphoreType.DMA,
    ],
)


ROWS_PER_STEP = 1000
GRID = N // ROWS_PER_STEP


def _tc_body(pagg0, pagg1, pdeg0, pdeg1, x, wm, ws, out):
    i = pl.program_id(0)
    agg = pagg0[...] + pagg1[...]
    deg = pdeg0[...] + pdeg1[...]
    aggn = agg / jnp.maximum(deg, 1.0)
    h = jnp.dot(aggn, wm[...], preferred_element_type=jnp.float32)
    h = h + jnp.dot(x[...], ws[...], preferred_element_type=jnp.float32)
    h = jnp.maximum(h, 0.0)
    part = jnp.sum(h, axis=0, keepdims=True) * jnp.float32(1.0 / N)

    @pl.when(i == 0)
    def _():
        out[...] = part

    @pl.when(i > 0)
    def _():
        out[...] += part


_tc_combine = pl.pallas_call(
    _tc_body,
    grid=(GRID,),
    in_specs=[
        pl.BlockSpec((ROWS_PER_STEP, D), lambda i: (i, 0)),
        pl.BlockSpec((ROWS_PER_STEP, D), lambda i: (i, 0)),
        pl.BlockSpec((ROWS_PER_STEP, D), lambda i: (i, 0)),
        pl.BlockSpec((ROWS_PER_STEP, D), lambda i: (i, 0)),
        pl.BlockSpec((ROWS_PER_STEP, D), lambda i: (i, 0)),
        pl.BlockSpec((D, D), lambda i: (0, 0)),
        pl.BlockSpec((D, D), lambda i: (0, 0)),
    ],
    out_specs=pl.BlockSpec((1, D), lambda i: (0, 0)),
    out_shape=jax.ShapeDtypeStruct((1, D), jnp.float32),
)


@jax.jit
def kernel(x, edge_index, W_msg, W_self):
    e4 = edge_index.reshape(2, NW, NBLK, BLK).transpose(1, 2, 0, 3)
    e4 = jnp.pad(e4, ((0, 0), (0, NBLK_PAD - NBLK), (0, 0), (0, 0)))
    zrows = jnp.zeros((BLK, D), jnp.float32)
    ones = jnp.ones((BLK, D), jnp.float32)
    pagg, pdeg = _sc_scatter(x, e4, zrows, ones)
    pooled = _tc_combine(pagg[0, :N], pagg[1, :N], pdeg[0, :N], pdeg[1, :N],
                         x, W_msg, W_self)
    return pooled.reshape(D)
